# baseline (speedup 1.0000x reference)
import functools
import jax
import jax.numpy as jnp
from jax import lax
from jax.experimental import pallas as pl
from jax.experimental.pallas import tpu as pltpu

_FRAC = 0.36787944117144233


def _gate_body(k_const, x_ref, o_ref, bits_ref, h16_ref):
    kk = jnp.int32(k_const)
    bits = lax.bitcast_convert_type(x_ref[...], jnp.int32) & jnp.int32(0x7FFFFFFF)
    bits_ref[...] = bits
    h16_ref[...] = lax.shift_right_logical(bits, 16).astype(jnp.int16)

    def cnt16(c16):
        m = (h16_ref[...] >= c16).astype(jnp.int16)
        return jnp.sum(jnp.sum(m, axis=0).astype(jnp.int32))

    def stp1(i, p):
        t = p | (jnp.int32(1) << (jnp.int32(14) - i))
        cnt = cnt16(t.astype(jnp.int16))
        return lax.select(cnt >= kk, t, p)

    p_hi = lax.fori_loop(0, 15, stp1, jnp.int32(0))

    hi = lax.shift_right_logical(bits_ref[...], 16)
    low_q = lax.shift_left(bits_ref[...] & jnp.int32(0xFFFF), 15)
    bits_ref[...] = jnp.where(
        hi == p_hi, low_q, jnp.where(hi > p_hi, jnp.int32(0x7FFFFFFF), jnp.int32(0))
    )
    h16_ref[...] = lax.shift_right_logical(bits_ref[...], 16).astype(jnp.int16)

    def stp2(i, p):
        t = p | (jnp.int32(1) << (jnp.int32(30) - i))
        cnt = cnt16(lax.shift_right_logical(t, 16).astype(jnp.int16))
        return lax.select(cnt >= kk, t, p)

    p_q = lax.fori_loop(0, 15, stp2, jnp.int32(0))

    t = p_q | (jnp.int32(1) << 15)
    cnt = jnp.sum((bits_ref[...] >= t).astype(jnp.int32))
    p_q = lax.select(cnt >= kk, t, p_q)

    p_full = lax.shift_left(p_hi, 16) | lax.shift_right_logical(p_q, 15)
    abs_bits = lax.bitcast_convert_type(x_ref[...], jnp.int32) & jnp.int32(0x7FFFFFFF)
    o_ref[...] = jnp.where(abs_bits >= p_full, x_ref[...], jnp.float32(0.0))


def kernel(x):
    n = x.size
    k = max(1, int(n * _FRAC))
    if k >= n:
        return x
    return pl.pallas_call(
        functools.partial(_gate_body, k),
        out_shape=jax.ShapeDtypeStruct(x.shape, x.dtype),
        scratch_shapes=[
            pltpu.VMEM(x.shape, jnp.int32),
            pltpu.VMEM(x.shape, jnp.int16),
        ],
    )(x)


# speedup vs baseline: 1.7208x; 1.7208x over previous
import functools
import jax
import jax.numpy as jnp
from jax import lax
from jax.experimental import pallas as pl
from jax.experimental.pallas import tpu as pltpu

_FRAC = 0.36787944117144233


def _gate_body(k_const, x_ref, o_ref, bits_ref):
    kk = jnp.int32(k_const)
    bits = lax.bitcast_convert_type(x_ref[...], jnp.int32) & jnp.int32(0x7FFFFFFF)
    bits_ref[...] = bits

    def step(i, p):
        cand = p | (jnp.int32(1) << (jnp.int32(30) - i))
        m = jnp.where(bits_ref[...] >= cand, jnp.int32(1), jnp.int32(0))
        cnt = jnp.sum(jnp.sum(m, axis=0))
        return lax.select(cnt >= kk, cand, p)

    p = lax.fori_loop(0, 31, step, jnp.int32(0))
    o_ref[...] = jnp.where(bits_ref[...] >= p, x_ref[...], jnp.float32(0.0))


def kernel(x):
    n = x.size
    k = max(1, int(n * _FRAC))
    if k >= n:
        return x
    return pl.pallas_call(
        functools.partial(_gate_body, k),
        out_shape=jax.ShapeDtypeStruct(x.shape, x.dtype),
        scratch_shapes=[pltpu.VMEM(x.shape, jnp.int32)],
    )(x)


# speedup vs baseline: 1.8554x; 1.0782x over previous
import functools
import jax
import jax.numpy as jnp
from jax import lax
from jax.experimental import pallas as pl
from jax.experimental.pallas import tpu as pltpu

_FRAC = 0.36787944117144233


def _gate_body(k_const, x_ref, o_ref, bits_ref, h16_ref):
    kk = jnp.int32(k_const)
    bits = lax.bitcast_convert_type(x_ref[...], jnp.int32) & jnp.int32(0x7FFFFFFF)
    bits_ref[...] = bits
    h16_ref[...] = lax.shift_right_logical(bits, 16).astype(jnp.int16)

    def cnt16(c16):
        m = jnp.where(h16_ref[...] >= c16, jnp.int16(1), jnp.int16(0))
        s = m[0:16] + m[16:32] + m[32:48] + m[48:64]
        s = s + m[64:80] + m[80:96] + m[96:112] + m[112:128]
        s32 = s.astype(jnp.int32)
        return jnp.sum(jnp.sum(s32, axis=0))

    def stp1(i, p):
        t = p | (jnp.int32(1) << (jnp.int32(14) - i))
        cnt = cnt16(t.astype(jnp.int16))
        return lax.select(cnt >= kk, t, p)

    p_hi = lax.fori_loop(0, 15, stp1, jnp.int32(0))

    hi = lax.shift_right_logical(bits_ref[...], 16)
    low_q = lax.shift_left(bits_ref[...] & jnp.int32(0xFFFF), 15)
    bits_ref[...] = jnp.where(
        hi == p_hi, low_q, jnp.where(hi > p_hi, jnp.int32(0x7FFFFFFF), jnp.int32(0))
    )
    h16_ref[...] = lax.shift_right_logical(bits_ref[...], 16).astype(jnp.int16)

    def stp2(i, p):
        t = p | (jnp.int32(1) << (jnp.int32(30) - i))
        cnt = cnt16(lax.shift_right_logical(t, 16).astype(jnp.int16))
        return lax.select(cnt >= kk, t, p)

    p_q = lax.fori_loop(0, 15, stp2, jnp.int32(0))

    t = p_q | (jnp.int32(1) << 15)
    m = jnp.where(bits_ref[...] >= t, jnp.int32(1), jnp.int32(0))
    cnt = jnp.sum(jnp.sum(m, axis=0))
    p_q = lax.select(cnt >= kk, t, p_q)

    p_full = lax.shift_left(p_hi, 16) | lax.shift_right_logical(p_q, 15)
    abs_bits = lax.bitcast_convert_type(x_ref[...], jnp.int32) & jnp.int32(0x7FFFFFFF)
    o_ref[...] = jnp.where(abs_bits >= p_full, x_ref[...], jnp.float32(0.0))


def kernel(x):
    n = x.size
    k = max(1, int(n * _FRAC))
    if k >= n:
        return x
    return pl.pallas_call(
        functools.partial(_gate_body, k),
        out_shape=jax.ShapeDtypeStruct(x.shape, x.dtype),
        scratch_shapes=[
            pltpu.VMEM(x.shape, jnp.int32),
            pltpu.VMEM(x.shape, jnp.int16),
        ],
    )(x)
